# Initial kernel scaffold; baseline (speedup 1.0000x reference)
import jax
import jax.numpy as jnp
from jax.experimental import pallas as pl


def kernel(x, edge_index, W_lin, b_lin, W_att, b_att):
    raise NotImplementedError("write your pallas kernel here")


# speedup vs baseline: 10.3592x; 10.3592x over previous
import functools

import jax
import jax.numpy as jnp
from jax import lax
from jax.experimental import pallas as pl
from jax.experimental.pallas import tpu as pltpu
from jax.experimental.pallas import tpu_sc as plsc

N = 10000
E = 320000
D = 128
NC = 2
NS = 16
NW = NC * NS
L = 16
_SC_PARAMS = pltpu.CompilerParams(
    needs_layout_passes=False, use_tc_tiling_on_sc=False)
EC = E // NW
NBLK = EC // L
RPT = N // NS


def _lin_body(x_ref, wt_ref, b_ref, wa_ref, h_ref, s_ref):
    h = jnp.dot(x_ref[...], wt_ref[...], preferred_element_type=jnp.float32)
    h = h + b_ref[...]
    h_ref[...] = h
    s_ref[...] = jnp.dot(h, wa_ref[...], preferred_element_type=jnp.float32)


def _linear(x, wt, b, wa):
    blk = 1000
    grid = N // blk
    return pl.pallas_call(
        _lin_body,
        grid=(grid,),
        in_specs=[
            pl.BlockSpec((blk, D), lambda i: (i, 0)),
            pl.BlockSpec((D, D), lambda i: (0, 0)),
            pl.BlockSpec((1, D), lambda i: (0, 0)),
            pl.BlockSpec((D, 8), lambda i: (0, 0)),
        ],
        out_specs=[
            pl.BlockSpec((blk, D), lambda i: (i, 0)),
            pl.BlockSpec((blk, 8), lambda i: (i, 0)),
        ],
        out_shape=[
            jax.ShapeDtypeStruct((N, D), jnp.float32),
            jax.ShapeDtypeStruct((N, 8), jnp.float32),
        ],
    )(x, wt, b, wa)


def _edge_scalar_body(rows_hbm, cols_hbm, s1_hbm, s2_hbm,
                      p_hbm, zpart_hbm, degpart_hbm,
                      rows_v, cols_v, s1_v, s2_v, p_v, z_v, deg_v):
    cid = lax.axis_index("c")
    sid = lax.axis_index("s")
    wid = sid * NC + cid
    base = wid * EC
    pltpu.sync_copy(rows_hbm.at[pl.ds(base, EC)], rows_v)
    pltpu.sync_copy(cols_hbm.at[pl.ds(base, EC)], cols_v)
    pltpu.sync_copy(s1_hbm, s1_v)
    pltpu.sync_copy(s2_hbm, s2_v)

    zeros = jnp.zeros((L,), jnp.float32)

    def zinit(i, carry):
        z_v[pl.ds(i * L, L)] = zeros
        deg_v[pl.ds(i * L, L)] = zeros
        return carry

    lax.fori_loop(0, N // L, zinit, 0)

    ones = jnp.ones((L,), jnp.float32)

    def ebody(i, carry):
        r = rows_v[pl.ds(i * L, L)]
        c = cols_v[pl.ds(i * L, L)]
        a = plsc.load_gather(s1_v, [r]) + plsc.load_gather(s2_v, [c])
        e = jnp.maximum(a, a * 0.2)
        p = jnp.exp(e)
        p_v[pl.ds(i * L, L)] = p
        plsc.addupdate_scatter(z_v, [r], p)
        plsc.addupdate_scatter(deg_v, [c], ones)
        return carry

    lax.fori_loop(0, NBLK, ebody, 0)

    pltpu.sync_copy(p_v, p_hbm.at[pl.ds(base, EC)])
    pltpu.sync_copy(z_v, zpart_hbm.at[wid])
    pltpu.sync_copy(deg_v, degpart_hbm.at[wid])


def _edge_scalar(rows, cols, s1, s2):
    mesh = plsc.VectorSubcoreMesh(
        core_axis_name="c", subcore_axis_name="s", num_cores=NC, num_subcores=NS)
    fn = pl.kernel(
        _edge_scalar_body,
        out_type=[
            jax.ShapeDtypeStruct((E,), jnp.float32),
            jax.ShapeDtypeStruct((NW, N), jnp.float32),
            jax.ShapeDtypeStruct((NW, N), jnp.float32),
        ],
        mesh=mesh,
        scratch_types=[
            pltpu.VMEM((EC,), jnp.int32),
            pltpu.VMEM((EC,), jnp.int32),
            pltpu.VMEM((N,), jnp.float32),
            pltpu.VMEM((N,), jnp.float32),
            pltpu.VMEM((EC,), jnp.float32),
            pltpu.VMEM((N,), jnp.float32),
            pltpu.VMEM((N,), jnp.float32),
        ],
        compiler_params=_SC_PARAMS,
    )
    return fn(rows, cols, s1, s2)


def _col_sums(m):
    ones = jnp.ones((m.shape[0], 1), jnp.float32)
    return jax.lax.dot_general(m, ones, (((0,), (0,)), ((), ())),
                               preferred_element_type=jnp.float32)


def _norm_body(h_ref, zp_ref, g_ref):
    z = _col_sums(zp_ref[0])
    g_ref[...] = h_ref[...] / (z + 1e-16)


def _normalize(h, zpart):
    blk = 1000
    zpart3 = zpart.reshape(NW, N // blk, blk).transpose(1, 0, 2)
    return pl.pallas_call(
        _norm_body,
        grid=(N // blk,),
        in_specs=[
            pl.BlockSpec((blk, D), lambda i: (i, 0)),
            pl.BlockSpec((1, NW, blk), lambda i: (i, 0, 0)),
        ],
        out_specs=pl.BlockSpec((blk, D), lambda i: (i, 0)),
        out_shape=jax.ShapeDtypeStruct((N, D), jnp.float32),
    )(h, zpart3)


def _agg_body(rows_hbm, cols_hbm, p_hbm, g_hbm, acc_hbm,
              rows_v, cols_v, p_v, gbuf, zrow, acc_sh, gsem):
    cid = lax.axis_index("c")
    sid = lax.axis_index("s")
    wid = sid * NC + cid
    base = wid * EC
    pltpu.sync_copy(rows_hbm.at[pl.ds(base, EC)], rows_v)
    pltpu.sync_copy(cols_hbm.at[pl.ds(base, EC)], cols_v)
    pltpu.sync_copy(p_hbm.at[pl.ds(base, EC)], p_v)

    zeros = jnp.zeros((L,), jnp.float32)

    def zrow_init(i, carry):
        zrow[0, pl.ds(i * L, L)] = zeros
        return carry

    lax.fori_loop(0, D // L, zrow_init, 0)

    def acc_init(i, carry):
        pltpu.sync_copy(zrow, acc_sh.at[pl.ds(sid * RPT + i, 1)])
        return carry

    lax.fori_loop(0, RPT, acc_init, 0)
    plsc.subcore_barrier()

    def ebody(i, carry):
        r = rows_v[pl.ds(i * L, L)]
        c = cols_v[pl.ds(i * L, L)]
        pltpu.async_copy(g_hbm.at[r], gbuf, gsem).wait()
        for e in range(L):
            pb = plsc.load_gather(p_v, [jnp.full((L,), i * L + e, jnp.int32)])
            for j in range(D // L):
                gbuf[e, pl.ds(j * L, L)] = gbuf[e, pl.ds(j * L, L)] * pb
        pltpu.sync_copy(gbuf, acc_sh.at[c], add=True)
        return carry

    lax.fori_loop(0, NBLK, ebody, 0)
    plsc.subcore_barrier()

    pltpu.sync_copy(acc_sh.at[pl.ds(sid * RPT, RPT)],
                    acc_hbm.at[cid, pl.ds(sid * RPT, RPT)])


def _aggregate(rows, cols, p, g):
    mesh = plsc.VectorSubcoreMesh(
        core_axis_name="c", subcore_axis_name="s", num_cores=NC, num_subcores=NS)
    fn = pl.kernel(
        _agg_body,
        out_type=jax.ShapeDtypeStruct((NC, N, D), jnp.float32),
        mesh=mesh,
        scratch_types=[
            pltpu.VMEM((EC,), jnp.int32),
            pltpu.VMEM((EC,), jnp.int32),
            pltpu.VMEM((EC,), jnp.float32),
            pltpu.VMEM((L, D), jnp.float32),
            pltpu.VMEM((1, D), jnp.float32),
            pltpu.VMEM_SHARED((N, D), jnp.float32),
            pltpu.SemaphoreType.DMA,
        ],
        compiler_params=_SC_PARAMS,
    )
    return fn(rows, cols, p, g)


def _combine_body(acc_ref, dp_ref, h_ref, o_ref):
    deg = _col_sums(dp_ref[0])
    o_ref[...] = jnp.where(deg > 0.0, acc_ref[0] + acc_ref[1], h_ref[...])


def _combine(acc, degpart, h):
    blk = 1000
    dp3 = degpart.reshape(NW, N // blk, blk).transpose(1, 0, 2)
    return pl.pallas_call(
        _combine_body,
        grid=(N // blk,),
        in_specs=[
            pl.BlockSpec((NC, blk, D), lambda i: (0, i, 0)),
            pl.BlockSpec((1, NW, blk), lambda i: (i, 0, 0)),
            pl.BlockSpec((blk, D), lambda i: (i, 0)),
        ],
        out_specs=pl.BlockSpec((blk, D), lambda i: (i, 0)),
        out_shape=jax.ShapeDtypeStruct((N, D), jnp.float32),
    )(acc, dp3, h)


def kernel(x, edge_index, W_lin, b_lin, W_att, b_att):
    rows = edge_index[0]
    cols = edge_index[1]
    wt = W_lin.T
    b = b_lin.reshape(1, D)
    wa1 = W_att[0, :D]
    wa2 = W_att[0, D:]
    wa = jnp.zeros((D, 8), jnp.float32).at[:, 0].set(wa1).at[:, 1].set(wa2)
    ba = jnp.zeros((1, 8), jnp.float32).at[0, 0].set(b_att[0])

    h, s = _linear(x, wt, b, wa)
    s = s + ba
    s1 = s[:, 0]
    s2 = s[:, 1]

    p, zpart, degpart = _edge_scalar(rows, cols, s1, s2)
    g = _normalize(h, zpart)
    acc = _aggregate(rows, cols, p, g)
    return _combine(acc, degpart, h)


# speedup vs baseline: 28.2623x; 2.7282x over previous
import functools

import jax
import jax.numpy as jnp
from jax import lax
from jax.experimental import pallas as pl
from jax.experimental.pallas import tpu as pltpu
from jax.experimental.pallas import tpu_sc as plsc

N = 10000
E = 320000
D = 128
NC = 2
NS = 16
NW = NC * NS
L = 16
_SC_PARAMS = pltpu.CompilerParams(
    needs_layout_passes=False, use_tc_tiling_on_sc=False)
EC = E // NW
NBLK = EC // L
RPT = N // NS


def _lin_body(x_ref, wt_ref, b_ref, wa_ref, h_ref, s_ref):
    h = jnp.dot(x_ref[...], wt_ref[...], preferred_element_type=jnp.float32)
    h = h + b_ref[...]
    h_ref[...] = h
    s_ref[...] = jnp.dot(h, wa_ref[...], preferred_element_type=jnp.float32)


def _linear(x, wt, b, wa):
    blk = 1000
    grid = N // blk
    return pl.pallas_call(
        _lin_body,
        grid=(grid,),
        in_specs=[
            pl.BlockSpec((blk, D), lambda i: (i, 0)),
            pl.BlockSpec((D, D), lambda i: (0, 0)),
            pl.BlockSpec((1, D), lambda i: (0, 0)),
            pl.BlockSpec((D, 8), lambda i: (0, 0)),
        ],
        out_specs=[
            pl.BlockSpec((blk, D), lambda i: (i, 0)),
            pl.BlockSpec((blk, 8), lambda i: (i, 0)),
        ],
        out_shape=[
            jax.ShapeDtypeStruct((N, D), jnp.float32),
            jax.ShapeDtypeStruct((N, 8), jnp.float32),
        ],
    )(x, wt, b, wa)


def _edge_scalar_body(rows_hbm, cols_hbm, s1_hbm, s2_hbm,
                      p_hbm, zpart_hbm, degpart_hbm,
                      rows_v, cols_v, s1_v, s2_v, p_v, z_v, deg_v):
    cid = lax.axis_index("c")
    sid = lax.axis_index("s")
    wid = sid * NC + cid
    base = wid * EC
    pltpu.sync_copy(rows_hbm.at[pl.ds(base, EC)], rows_v)
    pltpu.sync_copy(cols_hbm.at[pl.ds(base, EC)], cols_v)
    pltpu.sync_copy(s1_hbm, s1_v)
    pltpu.sync_copy(s2_hbm, s2_v)

    zeros = jnp.zeros((L,), jnp.float32)

    def zinit(i, carry):
        z_v[pl.ds(i * L, L)] = zeros
        deg_v[pl.ds(i * L, L)] = zeros
        return carry

    lax.fori_loop(0, N // L, zinit, 0)

    ones = jnp.ones((L,), jnp.float32)

    def ebody(i, carry):
        r = rows_v[pl.ds(i * L, L)]
        c = cols_v[pl.ds(i * L, L)]
        a = plsc.load_gather(s1_v, [r]) + plsc.load_gather(s2_v, [c])
        e = jnp.maximum(a, a * 0.2)
        p = jnp.exp(e)
        p_v[pl.ds(i * L, L)] = p
        plsc.addupdate_scatter(z_v, [r], p)
        plsc.addupdate_scatter(deg_v, [c], ones)
        return carry

    lax.fori_loop(0, NBLK, ebody, 0)

    pltpu.sync_copy(p_v, p_hbm.at[pl.ds(base, EC)])
    pltpu.sync_copy(z_v, zpart_hbm.at[wid])
    pltpu.sync_copy(deg_v, degpart_hbm.at[wid])


def _edge_scalar(rows, cols, s1, s2):
    mesh = plsc.VectorSubcoreMesh(
        core_axis_name="c", subcore_axis_name="s", num_cores=NC, num_subcores=NS)
    fn = pl.kernel(
        _edge_scalar_body,
        out_type=[
            jax.ShapeDtypeStruct((E,), jnp.float32),
            jax.ShapeDtypeStruct((NW, N), jnp.float32),
            jax.ShapeDtypeStruct((NW, N), jnp.float32),
        ],
        mesh=mesh,
        scratch_types=[
            pltpu.VMEM((EC,), jnp.int32),
            pltpu.VMEM((EC,), jnp.int32),
            pltpu.VMEM((N,), jnp.float32),
            pltpu.VMEM((N,), jnp.float32),
            pltpu.VMEM((EC,), jnp.float32),
            pltpu.VMEM((N,), jnp.float32),
            pltpu.VMEM((N,), jnp.float32),
        ],
        compiler_params=_SC_PARAMS,
    )
    return fn(rows, cols, s1, s2)


def _col_sums(m):
    ones = jnp.ones((m.shape[0], 1), jnp.float32)
    return jax.lax.dot_general(m, ones, (((0,), (0,)), ((), ())),
                               preferred_element_type=jnp.float32)


def _norm_body(h_ref, zp_ref, g_ref):
    z = _col_sums(zp_ref[0])
    g_ref[...] = h_ref[...] / (z + 1e-16)


def _normalize(h, zpart):
    blk = 1000
    zpart3 = zpart.reshape(NW, N // blk, blk).transpose(1, 0, 2)
    return pl.pallas_call(
        _norm_body,
        grid=(N // blk,),
        in_specs=[
            pl.BlockSpec((blk, D), lambda i: (i, 0)),
            pl.BlockSpec((1, NW, blk), lambda i: (i, 0, 0)),
        ],
        out_specs=pl.BlockSpec((blk, D), lambda i: (i, 0)),
        out_shape=jax.ShapeDtypeStruct((N, D), jnp.float32),
    )(h, zpart3)


NB = 5
PF = 3
ZR = 125


def _agg_body(rows_hbm, cols_hbm, p_hbm, g_hbm, zeros_hbm, acc_hbm,
              rows_v, cols_v, p_v, gbuf, acc_sh,
              g0, g1, g2, g3, g4, s0, s1, s2, s3, s4):
    gsems = (g0, g1, g2, g3, g4)
    ssems = (s0, s1, s2, s3, s4)
    cid = lax.axis_index("c")
    sid = lax.axis_index("s")
    wid = sid * NC + cid
    base = wid * EC
    pltpu.sync_copy(rows_hbm.at[pl.ds(base, EC)], rows_v)
    pltpu.sync_copy(cols_hbm.at[pl.ds(base, EC)], cols_v)
    pltpu.sync_copy(p_hbm.at[pl.ds(base, EC)], p_v)

    pltpu.sync_copy(zeros_hbm, acc_sh.at[pl.ds(sid * RPT, RPT)])
    plsc.subcore_barrier()

    def g_start(i, b):
        r = rows_v[pl.ds(i * L, L)]
        pltpu.async_copy(g_hbm.at[r], gbuf.at[b], gsems[b])

    def g_wait(b):
        pltpu.make_async_copy(g_hbm.at[pl.ds(0, L)], gbuf.at[b], gsems[b]).wait()

    def s_start(i, b):
        c = cols_v[pl.ds(i * L, L)]
        pltpu.async_copy(gbuf.at[b], acc_sh.at[c], ssems[b], add=True)

    def s_wait(b):
        pltpu.make_async_copy(g_hbm.at[pl.ds(0, L)], gbuf.at[b], ssems[b]).wait()

    def scale(i, b):
        for e in range(L):
            pb = plsc.load_gather(p_v, [jnp.full((L,), i * L + e, jnp.int32)])
            for j in range(D // L):
                gbuf[b, e, pl.ds(j * L, L)] = gbuf[b, e, pl.ds(j * L, L)] * pb

    for b in range(PF):
        g_start(b, b)

    n_outer = NBLK // NB

    def outer(io, carry):
        for b in range(NB):
            i = io * NB + b
            nxt = i + PF
            bb = (b + PF) % NB
            if b < NB - PF:
                if b < 2:
                    @pl.when(io > 0)
                    def _():
                        s_wait(bb)
                else:
                    s_wait(bb)
                g_start(nxt, bb)
            else:
                @pl.when(io < n_outer - 1)
                def _():
                    s_wait(bb)
                    g_start(nxt, bb)
            g_wait(b)
            scale(i, b)
            s_start(i, b)
        return carry

    lax.fori_loop(0, n_outer, outer, 0)
    for b in range(NB):
        s_wait(b)
    plsc.subcore_barrier()

    pltpu.sync_copy(acc_sh.at[pl.ds(sid * RPT, RPT)],
                    acc_hbm.at[cid, pl.ds(sid * RPT, RPT)])


def _aggregate(rows, cols, p, g):
    mesh = plsc.VectorSubcoreMesh(
        core_axis_name="c", subcore_axis_name="s", num_cores=NC, num_subcores=NS)
    fn = pl.kernel(
        _agg_body,
        out_type=jax.ShapeDtypeStruct((NC, N, D), jnp.float32),
        mesh=mesh,
        scratch_types=[
            pltpu.VMEM((EC,), jnp.int32),
            pltpu.VMEM((EC,), jnp.int32),
            pltpu.VMEM((EC,), jnp.float32),
            pltpu.VMEM((NB, L, D), jnp.float32),
            pltpu.VMEM_SHARED((N, D), jnp.float32),
        ] + [pltpu.SemaphoreType.DMA] * (2 * NB),
        compiler_params=_SC_PARAMS,
    )
    return fn(rows, cols, p, g, jnp.zeros((RPT, D), jnp.float32))


def _combine_body(acc_ref, dp_ref, h_ref, o_ref):
    deg = _col_sums(dp_ref[0])
    o_ref[...] = jnp.where(deg > 0.0, acc_ref[0] + acc_ref[1], h_ref[...])


def _combine(acc, degpart, h):
    blk = 1000
    dp3 = degpart.reshape(NW, N // blk, blk).transpose(1, 0, 2)
    return pl.pallas_call(
        _combine_body,
        grid=(N // blk,),
        in_specs=[
            pl.BlockSpec((NC, blk, D), lambda i: (0, i, 0)),
            pl.BlockSpec((1, NW, blk), lambda i: (i, 0, 0)),
            pl.BlockSpec((blk, D), lambda i: (i, 0)),
        ],
        out_specs=pl.BlockSpec((blk, D), lambda i: (i, 0)),
        out_shape=jax.ShapeDtypeStruct((N, D), jnp.float32),
    )(acc, dp3, h)


def kernel(x, edge_index, W_lin, b_lin, W_att, b_att):
    rows = edge_index[0]
    cols = edge_index[1]
    wt = W_lin.T
    b = b_lin.reshape(1, D)
    wa1 = W_att[0, :D]
    wa2 = W_att[0, D:]
    wa = jnp.zeros((D, 8), jnp.float32).at[:, 0].set(wa1).at[:, 1].set(wa2)
    ba = jnp.zeros((1, 8), jnp.float32).at[0, 0].set(b_att[0])

    h, s = _linear(x, wt, b, wa)
    s = s + ba
    s1 = s[:, 0]
    s2 = s[:, 1]

    p, zpart, degpart = _edge_scalar(rows, cols, s1, s2)
    g = _normalize(h, zpart)
    acc = _aggregate(rows, cols, p, g)
    return _combine(acc, degpart, h)
